# Initial kernel scaffold; baseline (speedup 1.0000x reference)
import jax
import jax.numpy as jnp
from jax.experimental import pallas as pl


def kernel(logits_bottom, logits_top, lbl_bottom, lbl_top):
    raise NotImplementedError("write your pallas kernel here")


# speedup vs baseline: 30.6664x; 30.6664x over previous
import functools

import jax
import jax.numpy as jnp
from jax import lax
from jax.experimental import pallas as pl
from jax.experimental.pallas import tpu as pltpu
from jax.experimental.pallas import tpu_sc as plsc

IGNORE = 255
RATIO = 0.1
THRESH = 2.5

N, CB, H, W = 4, 19, 512, 512
CT = 2
BH = 128
TOTAL = N * H * W
NTILES = 32
CHUNK = TOTAL // NTILES
NBINS = 2048
INVW = 64.0


def _loss_stats_body(lb_ref, lt_ref, lblb_ref, lblt_ref, loss_ref, stats_ref):
    n = pl.program_id(0)
    h = pl.program_id(1)
    lblb = lblb_ref[0]
    lblt = lblt_ref[0]

    x = lb_ref[0, 0]
    m_b = x
    xsel_b = jnp.where(lblb == 0, x, 0.0)
    for c in range(1, CB):
        x = lb_ref[0, c]
        m_b = jnp.maximum(m_b, x)
        xsel_b = xsel_b + jnp.where(lblb == c, x, 0.0)
    s_b = jnp.exp(lb_ref[0, 0] - m_b)
    for c in range(1, CB):
        s_b = s_b + jnp.exp(lb_ref[0, c] - m_b)
    valid_b = (lblb != IGNORE).astype(jnp.float32)
    ce_b = (jnp.log(s_b) + m_b - xsel_b) * valid_b

    t0 = lt_ref[0, 0]
    t1 = lt_ref[0, 1]
    m_t = jnp.maximum(t0, t1)
    xsel_t = jnp.where(lblt == 0, t0, 0.0) + jnp.where(lblt == 1, t1, 0.0)
    s_t = jnp.exp(t0 - m_t) + jnp.exp(t1 - m_t)
    valid_t = (lblt != IGNORE).astype(jnp.float32)
    ce_t = (jnp.log(s_t) + m_t - xsel_t) * valid_t

    hier = (m_b - m_t) ** 2
    loss = ce_t + hier + ce_b
    loss_ref[0] = loss

    so = (lblt == 1).astype(jnp.float32)
    sl = (lblt == 0).astype(jnp.float32)
    hard = (loss > THRESH).astype(jnp.float32)

    def rsum(v):
        return jnp.sum(v, axis=0, keepdims=True)

    rows = jnp.concatenate([
        rsum(so), rsum(loss * so), rsum(so * hard), rsum(loss * so * hard),
        rsum(sl), rsum(loss * sl), rsum(sl * hard), rsum(loss * sl * hard),
        rsum(valid_b), jnp.zeros((7, W), jnp.float32),
    ], axis=0)

    @pl.when((n == 0) & (h == 0))
    def _():
        stats_ref[...] = jnp.zeros_like(stats_ref)

    stats_ref[...] += rows


def _loss_and_stats(logits_bottom, logits_top, lbl_bottom, lbl_top):
    return pl.pallas_call(
        _loss_stats_body,
        grid=(N, H // BH),
        in_specs=[
            pl.BlockSpec((1, CB, BH, W), lambda n, h: (n, 0, h, 0)),
            pl.BlockSpec((1, CT, BH, W), lambda n, h: (n, 0, h, 0)),
            pl.BlockSpec((1, BH, W), lambda n, h: (n, h, 0)),
            pl.BlockSpec((1, BH, W), lambda n, h: (n, h, 0)),
        ],
        out_specs=[
            pl.BlockSpec((1, BH, W), lambda n, h: (n, h, 0)),
            pl.BlockSpec((16, W), lambda n, h: (0, 0)),
        ],
        out_shape=[
            jax.ShapeDtypeStruct((N, H, W), jnp.float32),
            jax.ShapeDtypeStruct((16, W), jnp.float32),
        ],
    )(logits_bottom, logits_top, lbl_bottom, lbl_top)


def _sc_hist_body(loss_hbm, lbl_hbm, ocnt_hbm, osum_hbm,
                  loss_v, lbl_v, hcnt_v, hsum_v):
    wid = lax.axis_index("s") * 2 + lax.axis_index("c")
    base = wid * CHUNK
    pltpu.sync_copy(loss_hbm.at[pl.ds(base, CHUNK)], loss_v)
    pltpu.sync_copy(lbl_hbm.at[pl.ds(base, CHUNK)], lbl_v)

    zeros16 = jnp.zeros((16,), jnp.float32)

    def init(i, carry):
        hcnt_v[pl.ds(i * 16, 16)] = zeros16
        hsum_v[pl.ds(i * 16, 16)] = zeros16
        return carry

    lax.fori_loop(0, (2 * NBINS) // 16, init, 0)

    ones16 = jnp.ones((16,), jnp.float32)

    def body(i, carry):
        lv = loss_v[pl.ds(i * 16, 16)]
        gv = lbl_v[pl.ds(i * 16, 16)]
        b = jnp.minimum((lv * INVW).astype(jnp.int32), NBINS - 1)
        idx = gv * NBINS + b
        plsc.addupdate_scatter(hcnt_v, [idx], ones16)
        plsc.addupdate_scatter(hsum_v, [idx], lv)
        return carry

    lax.fori_loop(0, CHUNK // 16, body, 0)

    pltpu.sync_copy(hcnt_v, ocnt_hbm.at[wid])
    pltpu.sync_copy(hsum_v, osum_hbm.at[wid])


def _sc_hist(loss_flat, lbl_flat):
    mesh = plsc.VectorSubcoreMesh(core_axis_name="c", subcore_axis_name="s")
    fn = pl.kernel(
        _sc_hist_body,
        out_type=[
            jax.ShapeDtypeStruct((NTILES, 2 * NBINS), jnp.float32),
            jax.ShapeDtypeStruct((NTILES, 2 * NBINS), jnp.float32),
        ],
        mesh=mesh,
        scratch_types=[
            pltpu.VMEM((CHUNK,), jnp.float32),
            pltpu.VMEM((CHUNK,), jnp.int32),
            pltpu.VMEM((2 * NBINS,), jnp.float32),
            pltpu.VMEM((2 * NBINS,), jnp.float32),
        ],
        compiler_params=pltpu.CompilerParams(needs_layout_passes=False),
    )
    return fn(loss_flat, lbl_flat)


def _suffix_topk(hc, hs, n_min):
    S = hc
    d = 1
    while d < NBINS:
        S = S + jnp.concatenate(
            [S[:, d:], jnp.zeros((1, d), jnp.float32)], axis=1)
        d *= 2
    full = (S <= n_min).astype(jnp.float32)
    cnt_full = jnp.sum(full * hc)
    sum_full = jnp.sum(full * hs)
    above = S - hc
    part = ((S > n_min) & (above <= n_min)).astype(jnp.float32)
    avg = jnp.sum(part * hs / jnp.maximum(hc, 1.0))
    return sum_full + (n_min - cnt_full) * avg


def _finalize_body(stats_ref, hcnt_ref, hsum_ref, out_ref):
    stats = stats_ref[...]
    cnt_s = jnp.sum(stats[0:1, :])
    sum_s = jnp.sum(stats[1:2, :])
    hcnt_s = jnp.sum(stats[2:3, :])
    hsum_s = jnp.sum(stats[3:4, :])
    cnt_l = jnp.sum(stats[4:5, :])
    sum_l = jnp.sum(stats[5:6, :])
    hcnt_l = jnp.sum(stats[6:7, :])
    hsum_l = jnp.sum(stats[7:8, :])
    n_valid = jnp.sum(stats[8:9, :])
    n_min = jnp.floor(n_valid * RATIO)

    hc = jnp.sum(hcnt_ref[...], axis=0, keepdims=True)
    hs = jnp.sum(hsum_ref[...], axis=0, keepdims=True)
    topk_l = _suffix_topk(hc[:, :NBINS], hs[:, :NBINS], n_min)
    topk_s = _suffix_topk(hc[:, NBINS:], hs[:, NBINS:], n_min)

    def select(cnt, sum_all, cnt_hard, sum_hard, topk):
        sum_sel = jnp.where(
            cnt_hard < n_min,
            jnp.where(cnt <= n_min, sum_all, topk),
            sum_hard,
        )
        cnt_sel = jnp.where(
            cnt_hard < n_min,
            jnp.where(cnt <= n_min, cnt, n_min),
            cnt_hard,
        )
        return sum_sel, cnt_sel

    ss, cs = select(cnt_s, sum_s, hcnt_s, hsum_s, topk_s)
    sl_, cl_ = select(cnt_l, sum_l, hcnt_l, hsum_l, topk_l)
    out_ref[0, 0] = (ss + sl_) / (cs + cl_)


def _finalize(stats, hcnt, hsum):
    return pl.pallas_call(
        _finalize_body,
        out_specs=pl.BlockSpec(memory_space=pltpu.SMEM),
        out_shape=jax.ShapeDtypeStruct((1, 1), jnp.float32),
    )(stats, hcnt, hsum)


def kernel(logits_bottom, logits_top, lbl_bottom, lbl_top):
    loss, stats = _loss_and_stats(logits_bottom, logits_top,
                                  lbl_bottom, lbl_top)
    hcnt, hsum = _sc_hist(loss.reshape(-1), lbl_top.reshape(-1))
    out = _finalize(stats, hcnt, hsum)
    return out[0, 0]


# speedup vs baseline: 31.9296x; 1.0412x over previous
import functools

import jax
import jax.numpy as jnp
from jax import lax
from jax.experimental import pallas as pl
from jax.experimental.pallas import tpu as pltpu
from jax.experimental.pallas import tpu_sc as plsc

IGNORE = 255
RATIO = 0.1
THRESH = 2.5

N, CB, H, W = 4, 19, 512, 512
CT = 2
BH = 128
TOTAL = N * H * W
NTILES = 32
CHUNK = TOTAL // NTILES
NBINS = 2048
INVW = NBINS / THRESH
UNROLL = 8


def _loss_stats_body(lb_ref, lt_ref, lblb_ref, lblt_ref, loss_ref, stats_ref):
    n = pl.program_id(0)
    h = pl.program_id(1)
    lblb = lblb_ref[0]
    lblt = lblt_ref[0]

    x = lb_ref[0, 0]
    m_b = x
    xsel_b = jnp.where(lblb == 0, x, 0.0)
    for c in range(1, CB):
        x = lb_ref[0, c]
        m_b = jnp.maximum(m_b, x)
        xsel_b = xsel_b + jnp.where(lblb == c, x, 0.0)
    s_b = jnp.exp(lb_ref[0, 0] - m_b)
    for c in range(1, CB):
        s_b = s_b + jnp.exp(lb_ref[0, c] - m_b)
    valid_b = (lblb != IGNORE).astype(jnp.float32)
    ce_b = (jnp.log(s_b) + m_b - xsel_b) * valid_b

    t0 = lt_ref[0, 0]
    t1 = lt_ref[0, 1]
    m_t = jnp.maximum(t0, t1)
    xsel_t = jnp.where(lblt == 0, t0, 0.0) + jnp.where(lblt == 1, t1, 0.0)
    s_t = jnp.exp(t0 - m_t) + jnp.exp(t1 - m_t)
    valid_t = (lblt != IGNORE).astype(jnp.float32)
    ce_t = (jnp.log(s_t) + m_t - xsel_t) * valid_t

    hier = (m_b - m_t) ** 2
    loss = ce_t + hier + ce_b
    loss_ref[0] = loss

    so = (lblt == 1).astype(jnp.float32)
    sl = (lblt == 0).astype(jnp.float32)
    hard = (loss > THRESH).astype(jnp.float32)

    def rsum(v):
        return jnp.sum(v, axis=0, keepdims=True)

    rows = jnp.concatenate([
        rsum(so), rsum(loss * so), rsum(so * hard), rsum(loss * so * hard),
        rsum(sl), rsum(loss * sl), rsum(sl * hard), rsum(loss * sl * hard),
        rsum(valid_b), jnp.zeros((7, W), jnp.float32),
    ], axis=0)

    @pl.when((n == 0) & (h == 0))
    def _():
        stats_ref[...] = jnp.zeros_like(stats_ref)

    stats_ref[...] += rows


def _loss_and_stats(logits_bottom, logits_top, lbl_bottom, lbl_top):
    return pl.pallas_call(
        _loss_stats_body,
        grid=(N, H // BH),
        in_specs=[
            pl.BlockSpec((1, CB, BH, W), lambda n, h: (n, 0, h, 0)),
            pl.BlockSpec((1, CT, BH, W), lambda n, h: (n, 0, h, 0)),
            pl.BlockSpec((1, BH, W), lambda n, h: (n, h, 0)),
            pl.BlockSpec((1, BH, W), lambda n, h: (n, h, 0)),
        ],
        out_specs=[
            pl.BlockSpec((1, BH, W), lambda n, h: (n, h, 0)),
            pl.BlockSpec((16, W), lambda n, h: (0, 0)),
        ],
        out_shape=[
            jax.ShapeDtypeStruct((N, H, W), jnp.float32),
            jax.ShapeDtypeStruct((16, W), jnp.float32),
        ],
    )(logits_bottom, logits_top, lbl_bottom, lbl_top)


def _sc_hist_body(loss_hbm, lbl_hbm, ocnt_hbm, osum_hbm,
                  loss_v, lbl_v, hcnt_v, hsum_v):
    wid = lax.axis_index("s") * 2 + lax.axis_index("c")
    base = wid * CHUNK
    pltpu.sync_copy(loss_hbm.at[pl.ds(base, CHUNK)], loss_v)
    pltpu.sync_copy(lbl_hbm.at[pl.ds(base, CHUNK)], lbl_v)

    zeros16 = jnp.zeros((16,), jnp.float32)

    def init(i, carry):
        hcnt_v[pl.ds(i * 16, 16)] = zeros16
        hsum_v[pl.ds(i * 16, 16)] = zeros16
        return carry

    lax.fori_loop(0, (2 * NBINS) // 16, init, 0)

    ones16 = jnp.ones((16,), jnp.float32)

    def body(i, carry):
        for j in range(UNROLL):
            off = (i * UNROLL + j) * 16
            lv = loss_v[pl.ds(off, 16)]
            gv = lbl_v[pl.ds(off, 16)]
            keep = lv <= THRESH
            b = jnp.minimum((lv * INVW).astype(jnp.int32), NBINS - 1)
            idx = gv * NBINS + b
            plsc.addupdate_scatter(hcnt_v, [idx], ones16, mask=keep)
            plsc.addupdate_scatter(hsum_v, [idx], lv, mask=keep)
        return carry

    lax.fori_loop(0, CHUNK // (16 * UNROLL), body, 0)

    pltpu.sync_copy(hcnt_v, ocnt_hbm.at[wid])
    pltpu.sync_copy(hsum_v, osum_hbm.at[wid])


def _sc_hist(loss_flat, lbl_flat):
    mesh = plsc.VectorSubcoreMesh(core_axis_name="c", subcore_axis_name="s")
    fn = pl.kernel(
        _sc_hist_body,
        out_type=[
            jax.ShapeDtypeStruct((NTILES, 2 * NBINS), jnp.float32),
            jax.ShapeDtypeStruct((NTILES, 2 * NBINS), jnp.float32),
        ],
        mesh=mesh,
        scratch_types=[
            pltpu.VMEM((CHUNK,), jnp.float32),
            pltpu.VMEM((CHUNK,), jnp.int32),
            pltpu.VMEM((2 * NBINS,), jnp.float32),
            pltpu.VMEM((2 * NBINS,), jnp.float32),
        ],
        compiler_params=pltpu.CompilerParams(needs_layout_passes=False),
    )
    return fn(loss_flat, lbl_flat)


def _suffix_topk(hc, hs, k):
    S = hc
    d = 1
    while d < NBINS:
        S = S + jnp.concatenate(
            [S[:, d:], jnp.zeros((1, d), jnp.float32)], axis=1)
        d *= 2
    full = (S <= k).astype(jnp.float32)
    cnt_full = jnp.sum(full * hc)
    sum_full = jnp.sum(full * hs)
    above = S - hc
    part = ((S > k) & (above <= k)).astype(jnp.float32)
    avg = jnp.sum(part * hs / jnp.maximum(hc, 1.0))
    return sum_full + (k - cnt_full) * avg


def _finalize_body(stats_ref, hcnt_ref, hsum_ref, out_ref):
    stats = stats_ref[...]
    cnt_s = jnp.sum(stats[0:1, :])
    sum_s = jnp.sum(stats[1:2, :])
    hcnt_s = jnp.sum(stats[2:3, :])
    hsum_s = jnp.sum(stats[3:4, :])
    cnt_l = jnp.sum(stats[4:5, :])
    sum_l = jnp.sum(stats[5:6, :])
    hcnt_l = jnp.sum(stats[6:7, :])
    hsum_l = jnp.sum(stats[7:8, :])
    n_valid = jnp.sum(stats[8:9, :])
    n_min = jnp.floor(n_valid * RATIO)

    hc = jnp.sum(hcnt_ref[...], axis=0, keepdims=True)
    hs = jnp.sum(hsum_ref[...], axis=0, keepdims=True)
    topk_l = hsum_l + _suffix_topk(hc[:, :NBINS], hs[:, :NBINS],
                                   n_min - hcnt_l)
    topk_s = hsum_s + _suffix_topk(hc[:, NBINS:], hs[:, NBINS:],
                                   n_min - hcnt_s)

    def select(cnt, sum_all, cnt_hard, sum_hard, topk):
        sum_sel = jnp.where(
            cnt_hard < n_min,
            jnp.where(cnt <= n_min, sum_all, topk),
            sum_hard,
        )
        cnt_sel = jnp.where(
            cnt_hard < n_min,
            jnp.where(cnt <= n_min, cnt, n_min),
            cnt_hard,
        )
        return sum_sel, cnt_sel

    ss, cs = select(cnt_s, sum_s, hcnt_s, hsum_s, topk_s)
    sl_, cl_ = select(cnt_l, sum_l, hcnt_l, hsum_l, topk_l)
    out_ref[0, 0] = (ss + sl_) / (cs + cl_)


def _finalize(stats, hcnt, hsum):
    return pl.pallas_call(
        _finalize_body,
        out_specs=pl.BlockSpec(memory_space=pltpu.SMEM),
        out_shape=jax.ShapeDtypeStruct((1, 1), jnp.float32),
    )(stats, hcnt, hsum)


def kernel(logits_bottom, logits_top, lbl_bottom, lbl_top):
    loss, stats = _loss_and_stats(logits_bottom, logits_top,
                                  lbl_bottom, lbl_top)
    hcnt, hsum = _sc_hist(loss.reshape(-1), lbl_top.reshape(-1))
    out = _finalize(stats, hcnt, hsum)
    return out[0, 0]


# speedup vs baseline: 37.9689x; 1.1891x over previous
import functools

import jax
import jax.numpy as jnp
from jax import lax
from jax.experimental import pallas as pl
from jax.experimental.pallas import tpu as pltpu
from jax.experimental.pallas import tpu_sc as plsc

IGNORE = 255
RATIO = 0.1
THRESH = 2.5

N, CB, H, W = 4, 19, 512, 512
CT = 2
BH = 128
TOTAL = N * H * W
NTILES = 32
CHUNK = TOTAL // NTILES
NBINS = 2048
INVW = NBINS / THRESH
UNROLL = 8


def _loss_stats_body(lb_ref, lt_ref, lblb_ref, lblt_ref, idx_ref, stats_ref):
    n = pl.program_id(0)
    h = pl.program_id(1)
    lblb = lblb_ref[0]
    lblt = lblt_ref[0]

    x = lb_ref[0, 0]
    m_b = x
    xsel_b = jnp.where(lblb == 0, x, 0.0)
    for c in range(1, CB):
        x = lb_ref[0, c]
        m_b = jnp.maximum(m_b, x)
        xsel_b = jnp.where(lblb == c, x, xsel_b)
    s_b = jnp.exp(lb_ref[0, 0] - m_b)
    for c in range(1, CB):
        s_b = s_b + jnp.exp(lb_ref[0, c] - m_b)
    valid_b = (lblb != IGNORE).astype(jnp.float32)
    ce_b = (jnp.log(s_b) + m_b - xsel_b) * valid_b

    t0 = lt_ref[0, 0]
    t1 = lt_ref[0, 1]
    m_t = jnp.maximum(t0, t1)
    xsel_t = jnp.where(lblt == 1, t1, jnp.where(lblt == 0, t0, 0.0))
    s_t = jnp.exp(t0 - m_t) + jnp.exp(t1 - m_t)
    valid_t = (lblt != IGNORE).astype(jnp.float32)
    ce_t = (jnp.log(s_t) + m_t - xsel_t) * valid_t

    hier = (m_b - m_t) ** 2
    loss = ce_t + hier + ce_b

    b = jnp.minimum((loss * INVW).astype(jnp.int32), NBINS - 1)
    slot = jnp.where(lblt == 1, NBINS, 0) + b
    idx_ref[0] = jnp.where(loss <= THRESH, slot, -1)

    so = (lblt == 1).astype(jnp.float32)
    sl = (lblt == 0).astype(jnp.float32)
    hard = (loss > THRESH).astype(jnp.float32)

    def rsum(v):
        return jnp.sum(v, axis=0, keepdims=True)

    rows = jnp.concatenate([
        rsum(so), rsum(loss * so), rsum(so * hard), rsum(loss * so * hard),
        rsum(sl), rsum(loss * sl), rsum(sl * hard), rsum(loss * sl * hard),
        rsum(valid_b), jnp.zeros((7, W), jnp.float32),
    ], axis=0)

    @pl.when((n == 0) & (h == 0))
    def _():
        stats_ref[...] = jnp.zeros_like(stats_ref)

    stats_ref[...] += rows


def _loss_and_stats(logits_bottom, logits_top, lbl_bottom, lbl_top):
    return pl.pallas_call(
        _loss_stats_body,
        grid=(N, H // BH),
        in_specs=[
            pl.BlockSpec((1, CB, BH, W), lambda n, h: (n, 0, h, 0)),
            pl.BlockSpec((1, CT, BH, W), lambda n, h: (n, 0, h, 0)),
            pl.BlockSpec((1, BH, W), lambda n, h: (n, h, 0)),
            pl.BlockSpec((1, BH, W), lambda n, h: (n, h, 0)),
        ],
        out_specs=[
            pl.BlockSpec((1, BH, W), lambda n, h: (n, h, 0)),
            pl.BlockSpec((16, W), lambda n, h: (0, 0)),
        ],
        out_shape=[
            jax.ShapeDtypeStruct((N, H, W), jnp.int32),
            jax.ShapeDtypeStruct((16, W), jnp.float32),
        ],
    )(logits_bottom, logits_top, lbl_bottom, lbl_top)


def _sc_hist_body(idx_hbm, ocnt_hbm, idx_v, hcnt_v):
    wid = lax.axis_index("s") * 2 + lax.axis_index("c")
    base = wid * CHUNK
    pltpu.sync_copy(idx_hbm.at[pl.ds(base, CHUNK)], idx_v)

    zeros16 = jnp.zeros((16,), jnp.float32)

    def init(i, carry):
        hcnt_v[pl.ds(i * 16, 16)] = zeros16
        return carry

    lax.fori_loop(0, (2 * NBINS) // 16, init, 0)

    ones16 = jnp.ones((16,), jnp.float32)

    def body(i, carry):
        for j in range(UNROLL):
            off = (i * UNROLL + j) * 16
            iv = idx_v[pl.ds(off, 16)]
            keep = iv >= 0
            safe = jnp.maximum(iv, 0)
            plsc.addupdate_scatter(hcnt_v, [safe], ones16, mask=keep)
        return carry

    lax.fori_loop(0, CHUNK // (16 * UNROLL), body, 0)

    pltpu.sync_copy(hcnt_v, ocnt_hbm.at[wid])


def _sc_hist(idx_flat):
    mesh = plsc.VectorSubcoreMesh(core_axis_name="c", subcore_axis_name="s")
    fn = pl.kernel(
        _sc_hist_body,
        out_type=jax.ShapeDtypeStruct((NTILES, 2 * NBINS), jnp.float32),
        mesh=mesh,
        scratch_types=[
            pltpu.VMEM((CHUNK,), jnp.int32),
            pltpu.VMEM((2 * NBINS,), jnp.float32),
        ],
        compiler_params=pltpu.CompilerParams(needs_layout_passes=False),
    )
    return fn(idx_flat)


def _suffix_topk(hc, centers, k):
    S = hc
    d = 1
    while d < NBINS:
        S = S + jnp.concatenate(
            [S[:, d:], jnp.zeros((1, d), jnp.float32)], axis=1)
        d *= 2
    full = (S <= k).astype(jnp.float32)
    cnt_full = jnp.sum(full * hc)
    sum_full = jnp.sum(full * hc * centers)
    above = S - hc
    part = ((S > k) & (above <= k)).astype(jnp.float32)
    center_t = jnp.sum(part * centers)
    return sum_full + (k - cnt_full) * center_t


def _finalize_body(stats_ref, hcnt_ref, out_ref):
    stats = stats_ref[...]
    cnt_s = jnp.sum(stats[0:1, :])
    sum_s = jnp.sum(stats[1:2, :])
    hcnt_s = jnp.sum(stats[2:3, :])
    hsum_s = jnp.sum(stats[3:4, :])
    cnt_l = jnp.sum(stats[4:5, :])
    sum_l = jnp.sum(stats[5:6, :])
    hcnt_l = jnp.sum(stats[6:7, :])
    hsum_l = jnp.sum(stats[7:8, :])
    n_valid = jnp.sum(stats[8:9, :])
    n_min = jnp.floor(n_valid * RATIO)

    hc = jnp.sum(hcnt_ref[...], axis=0, keepdims=True)
    centers = (lax.broadcasted_iota(jnp.int32, (1, NBINS), 1)
               .astype(jnp.float32) + 0.5) / INVW
    topk_l = hsum_l + _suffix_topk(hc[:, :NBINS], centers, n_min - hcnt_l)
    topk_s = hsum_s + _suffix_topk(hc[:, NBINS:], centers, n_min - hcnt_s)

    def select(cnt, sum_all, cnt_hard, sum_hard, topk):
        sum_sel = jnp.where(
            cnt_hard < n_min,
            jnp.where(cnt <= n_min, sum_all, topk),
            sum_hard,
        )
        cnt_sel = jnp.where(
            cnt_hard < n_min,
            jnp.where(cnt <= n_min, cnt, n_min),
            cnt_hard,
        )
        return sum_sel, cnt_sel

    ss, cs = select(cnt_s, sum_s, hcnt_s, hsum_s, topk_s)
    sl_, cl_ = select(cnt_l, sum_l, hcnt_l, hsum_l, topk_l)
    out_ref[0, 0] = (ss + sl_) / (cs + cl_)


def _finalize(stats, hcnt):
    return pl.pallas_call(
        _finalize_body,
        out_specs=pl.BlockSpec(memory_space=pltpu.SMEM),
        out_shape=jax.ShapeDtypeStruct((1, 1), jnp.float32),
    )(stats, hcnt)


def kernel(logits_bottom, logits_top, lbl_bottom, lbl_top):
    idx, stats = _loss_and_stats(logits_bottom, logits_top,
                                 lbl_bottom, lbl_top)
    hcnt = _sc_hist(idx.reshape(-1))
    out = _finalize(stats, hcnt)
    return out[0, 0]
